# baseline (speedup 1.0000x reference)
import functools

import jax
import jax.numpy as jnp
from jax import lax
from jax.experimental import pallas as pl
from jax.experimental.pallas import tpu as pltpu
from jax.experimental.pallas import tpu_sc as plsc

NC = 2
NS = 16
NW = NC * NS
D = 64
CB = 128
S = 50

PREP_COLS = 2048


def _prep_block(in_ref, out_ref):
    t = in_ref[...].T
    out_ref[...] = jnp.concatenate([t, t], axis=1)


def _tpose_block(in_ref, out_ref):
    out_ref[0] = in_ref[0].T[:D, :]


@jax.jit
def _emb_lookup(xT, table):
    Btot = xT.shape[1]
    b_per_w = Btot // NW
    jblocks = b_per_w // CB
    n_units = S * jblocks

    mesh = plsc.VectorSubcoreMesh(core_axis_name="c", subcore_axis_name="s")

    @functools.partial(
        pl.kernel,
        out_type=jax.ShapeDtypeStruct((S, Btot, 2 * D), jnp.float32),
        mesh=mesh,
        scratch_types=[
            pltpu.VMEM((S, b_per_w), jnp.int32),
            pltpu.VMEM((2, CB, 2 * D), jnp.float32),
            pltpu.SemaphoreType.DMA,
            pltpu.SemaphoreType.DMA,
        ],
        compiler_params=pltpu.CompilerParams(use_tc_tiling_on_sc=False),
    )
    def emb(table_hbm, xT_hbm, out_hbm, idx_v, rowbuf, sem_g, sem_s):
        wid = lax.axis_index("s") * NC + lax.axis_index("c")
        base = wid * b_per_w
        pltpu.sync_copy(xT_hbm.at[:, pl.ds(base, b_per_w)], idx_v)

        def gather_desc(k, p):
            s = k // jblocks
            j = k % jblocks
            return pltpu.make_async_copy(
                table_hbm.at[idx_v.at[s, pl.ds(j * CB, CB)]],
                rowbuf.at[p],
                sem_g,
            )

        def store_desc(k, p):
            s = k // jblocks
            j = k % jblocks
            return pltpu.make_async_copy(
                rowbuf.at[p, :, pl.ds(0, D)],
                out_hbm.at[s, pl.ds(base + j * CB, CB), pl.ds(0, D)],
                sem_s,
            )

        gather_desc(0, 0).start()

        def group(g, carry):
            for p in range(2):
                k = g * 2 + p

                @pl.when(k + 1 < n_units)
                def _():
                    gather_desc(k + 1, 1 - p).start()

                gather_desc(k, p).wait()

                @pl.when(k >= 2)
                def _():
                    store_desc(k - 2, p).wait()

                store_desc(k, p).start()
            return carry

        lax.fori_loop(0, n_units // 2, group, 0)
        store_desc(n_units - 2, 0).wait()
        store_desc(n_units - 1, 1).wait()

    return emb(table, xT)


def kernel(x, Embeddings):
    B0, B1 = x.shape
    V = Embeddings.shape[0]

    prep = pl.pallas_call(
        _prep_block,
        grid=(pl.cdiv(V, PREP_COLS),),
        in_specs=[pl.BlockSpec((D, PREP_COLS), lambda g: (0, g))],
        out_specs=pl.BlockSpec((PREP_COLS, 2 * D), lambda g: (g, 0)),
        out_shape=jax.ShapeDtypeStruct((V, 2 * D), jnp.float32),
    )
    table_pad = prep(Embeddings.T)

    xT = x.T.astype(jnp.int32)
    out3 = _emb_lookup(xT, table_pad)

    return out3[:, :, :D].transpose(1, 0, 2)


# speedup vs baseline: 1.2709x; 1.2709x over previous
import functools

import jax
import jax.numpy as jnp
from jax import lax
from jax.experimental import pallas as pl
from jax.experimental.pallas import tpu as pltpu
from jax.experimental.pallas import tpu_sc as plsc

NC = 2
NS = 16
NW = NC * NS
D = 64
CB = 128
S = 50

PREP_COLS = 8192


def _prep_block(in_ref, out_ref):
    out_ref[:, :D] = in_ref[...].T


def _tpose_block(in_ref, out_ref):
    out_ref[0] = in_ref[0].T[:D, :]


@jax.jit
def _emb_lookup(xT, table):
    Btot = xT.shape[1]
    b_per_w = Btot // NW
    jblocks = b_per_w // CB
    n_units = S * jblocks

    mesh = plsc.VectorSubcoreMesh(core_axis_name="c", subcore_axis_name="s")

    @functools.partial(
        pl.kernel,
        out_type=jax.ShapeDtypeStruct((S, Btot, 2 * D), jnp.float32),
        mesh=mesh,
        scratch_types=[
            pltpu.VMEM((S, b_per_w), jnp.int32),
            pltpu.VMEM((2, CB, 2 * D), jnp.float32),
            pltpu.SemaphoreType.DMA,
            pltpu.SemaphoreType.DMA,
        ],
        compiler_params=pltpu.CompilerParams(use_tc_tiling_on_sc=False),
    )
    def emb(table_hbm, xT_hbm, out_hbm, idx_v, rowbuf, sem_g, sem_s):
        wid = lax.axis_index("s") * NC + lax.axis_index("c")
        base = wid * b_per_w
        pltpu.sync_copy(xT_hbm.at[:, pl.ds(base, b_per_w)], idx_v)

        def gather_desc(k, p):
            s = k // jblocks
            j = k % jblocks
            return pltpu.make_async_copy(
                table_hbm.at[idx_v.at[s, pl.ds(j * CB, CB)]],
                rowbuf.at[p],
                sem_g,
            )

        def store_desc(k, p):
            s = k // jblocks
            j = k % jblocks
            return pltpu.make_async_copy(
                rowbuf.at[p, :, pl.ds(0, D)],
                out_hbm.at[s, pl.ds(base + j * CB, CB), pl.ds(0, D)],
                sem_s,
            )

        gather_desc(0, 0).start()

        def group(g, carry):
            for p in range(2):
                k = g * 2 + p

                @pl.when(k + 1 < n_units)
                def _():
                    gather_desc(k + 1, 1 - p).start()

                gather_desc(k, p).wait()

                @pl.when(k >= 2)
                def _():
                    store_desc(k - 2, p).wait()

                store_desc(k, p).start()
            return carry

        lax.fori_loop(0, n_units // 2, group, 0)
        store_desc(n_units - 2, 0).wait()
        store_desc(n_units - 1, 1).wait()

    return emb(table, xT)


def kernel(x, Embeddings):
    B0, B1 = x.shape
    V = Embeddings.shape[0]

    prep = pl.pallas_call(
        _prep_block,
        grid=(pl.cdiv(V, PREP_COLS),),
        in_specs=[pl.BlockSpec((D, PREP_COLS), lambda g: (0, g))],
        out_specs=pl.BlockSpec((PREP_COLS, 2 * D), lambda g: (g, 0)),
        out_shape=jax.ShapeDtypeStruct((V, 2 * D), jnp.float32),
    )
    table_pad = prep(Embeddings.T)

    xT = x.T.astype(jnp.int32)
    out3 = _emb_lookup(xT, table_pad)

    return out3[:, :, :D].transpose(1, 0, 2)


# speedup vs baseline: 1.4624x; 1.1506x over previous
import functools

import jax
import jax.numpy as jnp
from jax import lax
from jax.experimental import pallas as pl
from jax.experimental.pallas import tpu as pltpu
from jax.experimental.pallas import tpu_sc as plsc

NC = 2
NS = 16
NW = NC * NS
D = 64
CB = 128
S = 50

PREP_COLS = 16384


def _prep_block(in_ref, out_ref):
    out_ref[:, :D] = in_ref[...].T


def _tpose_block(in_ref, out_ref):
    out_ref[0] = in_ref[0].T[:D, :]


@jax.jit
def _emb_lookup(xT, table):
    Btot = xT.shape[1]
    b_per_w = Btot // NW
    jblocks = b_per_w // CB
    n_units = S * jblocks

    mesh = plsc.VectorSubcoreMesh(core_axis_name="c", subcore_axis_name="s")

    @functools.partial(
        pl.kernel,
        out_type=jax.ShapeDtypeStruct((S, Btot, 2 * D), jnp.float32),
        mesh=mesh,
        scratch_types=[
            pltpu.VMEM((S, b_per_w), jnp.int32),
            pltpu.VMEM((2, CB, 2 * D), jnp.float32),
            pltpu.SemaphoreType.DMA,
            pltpu.SemaphoreType.DMA,
        ],
        compiler_params=pltpu.CompilerParams(use_tc_tiling_on_sc=False),
    )
    def emb(table_hbm, xT_hbm, out_hbm, idx_v, rowbuf, sem_g, sem_s):
        wid = lax.axis_index("s") * NC + lax.axis_index("c")
        base = wid * b_per_w
        pltpu.sync_copy(xT_hbm.at[:, pl.ds(base, b_per_w)], idx_v)

        def gather_desc(k, p):
            s = k // jblocks
            j = k % jblocks
            return pltpu.make_async_copy(
                table_hbm.at[idx_v.at[s, pl.ds(j * CB, CB)]],
                rowbuf.at[p],
                sem_g,
            )

        def store_desc(k, p):
            s = k // jblocks
            j = k % jblocks
            return pltpu.make_async_copy(
                rowbuf.at[p, :, pl.ds(0, D)],
                out_hbm.at[s, pl.ds(base + j * CB, CB), pl.ds(0, D)],
                sem_s,
            )

        gather_desc(0, 0).start()

        def group(g, carry):
            for p in range(2):
                k = g * 2 + p

                @pl.when(k + 1 < n_units)
                def _():
                    gather_desc(k + 1, 1 - p).start()

                gather_desc(k, p).wait()

                @pl.when(k >= 2)
                def _():
                    store_desc(k - 2, p).wait()

                store_desc(k, p).start()
            return carry

        lax.fori_loop(0, n_units // 2, group, 0)
        store_desc(n_units - 2, 0).wait()
        store_desc(n_units - 1, 1).wait()

    return emb(table, xT)


def kernel(x, Embeddings):
    B0, B1 = x.shape
    V = Embeddings.shape[0]

    prep = pl.pallas_call(
        _prep_block,
        grid=(pl.cdiv(V, PREP_COLS),),
        in_specs=[pl.BlockSpec((D, PREP_COLS), lambda g: (0, g))],
        out_specs=pl.BlockSpec((PREP_COLS, 2 * D), lambda g: (g, 0)),
        out_shape=jax.ShapeDtypeStruct((V, 2 * D), jnp.float32),
    )
    table_pad = prep(Embeddings.T)

    xT = x.T.astype(jnp.int32)
    out3 = _emb_lookup(xT, table_pad)

    return out3[:, :, :D].transpose(1, 0, 2)
